# baseline (speedup 1.0000x reference)
import functools

import jax
import jax.numpy as jnp
from jax import lax
from jax.experimental import pallas as pl
from jax.experimental.pallas import tpu as pltpu
from jax.experimental.pallas import tpu_sc as plsc

_QB = 2000
_BIG = 1e9


def _argmin_body(f_ref, q_ref, idx_ref, fm2_ref, x1n_ref, val_ref, blk_ref,
                 *, q_total):
    i = pl.program_id(0)
    nb = pl.num_programs(0)

    @pl.when(i == 0)
    def _():
        f = f_ref[...]
        fm2_ref[...] = -2.0 * f
        x1n_ref[...] = jnp.sum(f * f, axis=1, keepdims=True)

    q = q_ref[...]
    x2n = jnp.sum(q * q, axis=1)
    cross = lax.dot_general(
        fm2_ref[...], q, (((1,), (1,)), ((), ())),
        preferred_element_type=jnp.float32)
    d = (x1n_ref[...] + x2n[None, :]) + cross

    @pl.when(i == 0)
    def _():
        val_ref[...] = d
        blk_ref[...] = jnp.zeros_like(d)

    @pl.when(i > 0)
    def _():
        prev = val_ref[...]
        better = d < prev
        val_ref[...] = jnp.where(better, d, prev)
        blk_ref[...] = jnp.where(better, jnp.float32(i), blk_ref[...])

    @pl.when(i == nb - 1)
    def _():
        rv = val_ref[...]
        lmin = jnp.min(rv, axis=1, keepdims=True)
        lane = lax.broadcasted_iota(jnp.int32, rv.shape, 1).astype(jnp.float32)
        gidx = blk_ref[...] * jnp.float32(_QB) + lane
        sel = jnp.where(rv == lmin, gidx, _BIG)
        idx_ref[...] = jnp.min(sel, axis=1, keepdims=True).astype(jnp.int32)


def _argmin_tc(features, queue):
    b, d = features.shape
    q_total = queue.shape[0]
    nblocks = q_total // _QB
    idx = pl.pallas_call(
        functools.partial(_argmin_body, q_total=q_total),
        grid=(nblocks,),
        in_specs=[
            pl.BlockSpec((b, d), lambda i: (0, 0)),
            pl.BlockSpec((_QB, d), lambda i: (i, 0)),
        ],
        out_specs=pl.BlockSpec((b, 1), lambda i: (0, 0)),
        out_shape=jax.ShapeDtypeStruct((b, 1), jnp.int32),
        scratch_shapes=[
            pltpu.VMEM((b, d), jnp.float32),
            pltpu.VMEM((b, 1), jnp.float32),
            pltpu.VMEM((b, _QB), jnp.float32),
            pltpu.VMEM((b, _QB), jnp.float32),
        ],
    )(features, queue)
    return idx.reshape(b)


def _gather_sc(queue, idx):
    b = idx.shape[0]
    q_total, d = queue.shape
    nc, ns = 2, 16
    nw = nc * ns
    b_per_w = b // nw

    mesh = plsc.VectorSubcoreMesh(core_axis_name="c", subcore_axis_name="s")

    @functools.partial(
        pl.kernel, mesh=mesh,
        out_type=jax.ShapeDtypeStruct((b, d), jnp.float32),
        compiler_params=pltpu.CompilerParams(use_tc_tiling_on_sc=False),
        scratch_types=[
            pltpu.VMEM((b_per_w,), jnp.int32),
            pltpu.VMEM((b_per_w, d), jnp.float32),
            pltpu.SemaphoreType.DMA,
        ],
    )
    def gather(queue_hbm, idx_hbm, out_hbm, idx_v, rows_v, sem):
        wid = lax.axis_index("s") * nc + lax.axis_index("c")
        base = wid * b_per_w
        pltpu.sync_copy(idx_hbm.at[pl.ds(base, b_per_w)], idx_v)
        pltpu.async_copy(queue_hbm.at[idx_v], rows_v, sem).wait()
        pltpu.sync_copy(rows_v, out_hbm.at[pl.ds(base, b_per_w)])

    return gather(queue, idx)


def kernel(features, queue):
    idx = _argmin_tc(features, queue)
    return _gather_sc(queue, idx)


# speedup vs baseline: 1.2055x; 1.2055x over previous
import functools

import jax
import jax.numpy as jnp
from jax import lax
from jax.experimental import pallas as pl
from jax.experimental.pallas import tpu as pltpu
from jax.experimental.pallas import tpu_sc as plsc

_QB = 2000
_BIG = 1e9


def _argmin_body(f_ref, q_ref, idx_ref, fm2_ref, x1n_ref, val_ref, gidx_ref,
                 *, q_total):
    i = pl.program_id(0)
    nb = pl.num_programs(0)

    @pl.when(i == 0)
    def _():
        f = f_ref[...]
        fm2_ref[...] = -2.0 * f
        x1n_ref[...] = jnp.sum(f * f, axis=1, keepdims=True)

    q = q_ref[...]
    x2n = jnp.sum(q * q, axis=1)
    cross = lax.dot_general(
        fm2_ref[...], q, (((1,), (1,)), ((), ())),
        preferred_element_type=jnp.float32)
    d = (x1n_ref[...] + x2n[None, :]) + cross
    lmin = jnp.min(d, axis=1, keepdims=True)
    lane = lax.broadcasted_iota(jnp.int32, d.shape, 1).astype(jnp.float32)
    lidx = jnp.min(jnp.where(d == lmin, lane, _BIG),
                   axis=1, keepdims=True)
    gidx = lidx + jnp.float32(i * _QB)

    @pl.when(i == 0)
    def _():
        val_ref[...] = lmin
        gidx_ref[...] = gidx

    @pl.when(i > 0)
    def _():
        prev = val_ref[...]
        better = lmin < prev
        val_ref[...] = jnp.where(better, lmin, prev)
        gidx_ref[...] = jnp.where(better, gidx, gidx_ref[...])

    @pl.when(i == nb - 1)
    def _():
        idx_ref[...] = gidx_ref[...].astype(jnp.int32)


def _argmin_tc(features, queue):
    b, d = features.shape
    q_total = queue.shape[0]
    nblocks = q_total // _QB
    idx = pl.pallas_call(
        functools.partial(_argmin_body, q_total=q_total),
        grid=(nblocks,),
        in_specs=[
            pl.BlockSpec((b, d), lambda i: (0, 0)),
            pl.BlockSpec((_QB, d), lambda i: (i, 0)),
        ],
        out_specs=pl.BlockSpec((b, 1), lambda i: (0, 0)),
        out_shape=jax.ShapeDtypeStruct((b, 1), jnp.int32),
        scratch_shapes=[
            pltpu.VMEM((b, d), jnp.float32),
            pltpu.VMEM((b, 1), jnp.float32),
            pltpu.VMEM((b, 1), jnp.float32),
            pltpu.VMEM((b, 1), jnp.float32),
        ],
    )(features, queue)
    return idx.reshape(b)


def _gather_sc(queue, idx):
    b = idx.shape[0]
    q_total, d = queue.shape
    nc, ns = 2, 16
    nw = nc * ns
    b_per_w = b // nw

    mesh = plsc.VectorSubcoreMesh(core_axis_name="c", subcore_axis_name="s")

    @functools.partial(
        pl.kernel, mesh=mesh,
        out_type=jax.ShapeDtypeStruct((b, d), jnp.float32),
        compiler_params=pltpu.CompilerParams(use_tc_tiling_on_sc=False),
        scratch_types=[
            pltpu.VMEM((b_per_w,), jnp.int32),
            pltpu.VMEM((b_per_w, d), jnp.float32),
            pltpu.SemaphoreType.DMA,
        ],
    )
    def gather(queue_hbm, idx_hbm, out_hbm, idx_v, rows_v, sem):
        wid = lax.axis_index("s") * nc + lax.axis_index("c")
        base = wid * b_per_w
        pltpu.sync_copy(idx_hbm.at[pl.ds(base, b_per_w)], idx_v)
        pltpu.async_copy(queue_hbm.at[idx_v], rows_v, sem).wait()
        pltpu.sync_copy(rows_v, out_hbm.at[pl.ds(base, b_per_w)])

    return gather(queue, idx)


def kernel(features, queue):
    idx = _argmin_tc(features, queue)
    return _gather_sc(queue, idx)
